# Initial kernel scaffold; baseline (speedup 1.0000x reference)
import jax
import jax.numpy as jnp
from jax.experimental import pallas as pl


def kernel(x, neigh_orders):
    raise NotImplementedError("write your pallas kernel here")


# speedup vs baseline: 3.4633x; 3.4633x over previous
import functools

import jax
import jax.numpy as jnp
from jax import lax
from jax.experimental import pallas as pl
from jax.experimental.pallas import tpu as pltpu
from jax.experimental.pallas import tpu_sc as plsc

B, V, C = 4, 40962, 256
M = (V + 6) // 4
NW = 32
NPW = 352
NODES_PAD = NW * NPW
NB = 16
NCH = NPW // NB
ROWS = NB * 7
IDXW = NPW * 7
IDX_PAD = NW * IDXW
UN = 4
INV7 = float(1.0 / 7.0)


def _pool_body(x_hbm, no_hbm, out_hbm, idx_raw, idx4, gbuf, obuf,
               sem_g0, sem_g1, sem_w0, sem_w1):
    wid = lax.axis_index("s") * 2 + lax.axis_index("c")
    node_base = wid * NPW

    pltpu.sync_copy(no_hbm.at[pl.ds(wid * IDXW, IDXW)], idx_raw)

    def fill_b(b, _):
        def fill_i(i, _):
            v = idx_raw[pl.ds(i * 16, 16)]
            idx4[pl.ds(b * IDXW + i * 16, 16)] = v + b * V
            return 0
        return lax.fori_loop(0, IDXW // 16, fill_i, 0)
    lax.fori_loop(0, B, fill_b, 0)

    lane7 = 7 * lax.iota(jnp.int32, 16)
    sem_g = (sem_g0, sem_g1)
    sem_w = (sem_w0, sem_w1)

    def g_desc(b, c, par):
        idxsl = idx4.at[pl.ds(b * IDXW + c * ROWS, ROWS)]
        return pltpu.make_async_copy(x_hbm.at[idxsl], gbuf.at[par], sem_g[par])

    def w_desc(b, c, par):
        rowbase = b * NODES_PAD + node_base + c * NB
        return pltpu.make_async_copy(obuf.at[par],
                                     out_hbm.at[pl.ds(rowbase, NB)],
                                     sem_w[par])

    def compute_chunk(par):
        for oc in range(16):
            t = [lane7 + (112 * oc + k) for k in range(7)]
            rows = [lax.shift_right_logical(tk, 8) for tk in t]
            cols = [lax.bitwise_and(tk, 255) for tk in t]

            def nbody(ni, _):
                for u in range(UN):
                    n = ni * UN + u
                    roff = 7 * n
                    acc = plsc.load_gather(gbuf.at[par], [rows[0] + roff, cols[0]])
                    for k in range(1, 7):
                        acc = acc + plsc.load_gather(
                            gbuf.at[par], [rows[k] + roff, cols[k]])
                    obuf[par, n, pl.ds(oc * 16, 16)] = acc * INV7
                return 0
            lax.fori_loop(0, NB // UN, nbody, 0)

    def batch_body(b, _):
        g_desc(b, 0, 0).start()
        g_desc(b, 1, 1).start()

        def pair_body(i, _):
            for par in (0, 1):
                c = 2 * i + par
                g_desc(b, c, par).wait()

                @pl.when(i >= 1)
                def _():
                    w_desc(b, c - 2, par).wait()

                compute_chunk(par)
                w_desc(b, c, par).start()

                @pl.when(i < NCH // 2 - 1)
                def _():
                    g_desc(b, c + 2, par).start()
            return 0
        lax.fori_loop(0, NCH // 2, pair_body, 0)

        w_desc(b, NCH - 2, 0).wait()
        w_desc(b, NCH - 1, 1).wait()
        return 0
    lax.fori_loop(0, B, batch_body, 0)


@jax.jit
def _pool(x2, no_pad):
    mesh = plsc.VectorSubcoreMesh(core_axis_name="c", subcore_axis_name="s")
    f = pl.kernel(
        _pool_body,
        out_type=jax.ShapeDtypeStruct((B * NODES_PAD, C), jnp.float32),
        mesh=mesh,
        compiler_params=pltpu.CompilerParams(
            use_tc_tiling_on_sc=False, needs_layout_passes=False),
        scratch_types=[
            pltpu.VMEM((IDXW,), jnp.int32),
            pltpu.VMEM((B * IDXW,), jnp.int32),
            pltpu.VMEM((2, ROWS, C), jnp.float32),
            pltpu.VMEM((2, NB, C), jnp.float32),
            pltpu.SemaphoreType.DMA,
            pltpu.SemaphoreType.DMA,
            pltpu.SemaphoreType.DMA,
            pltpu.SemaphoreType.DMA,
        ],
    )
    return f(x2, no_pad)


def kernel(x, neigh_orders):
    x2 = x.reshape(B * V, C)
    no_pad = jnp.pad(neigh_orders[: M * 7], (0, IDX_PAD - M * 7))
    out = _pool(x2, no_pad)
    return out.reshape(B, NODES_PAD, C)[:, :M, :]


# speedup vs baseline: 3.4698x; 1.0019x over previous
import functools

import jax
import jax.numpy as jnp
from jax import lax
from jax.experimental import pallas as pl
from jax.experimental.pallas import tpu as pltpu
from jax.experimental.pallas import tpu_sc as plsc

B, V, C = 4, 40962, 256
M = (V + 6) // 4
NW = 32
NPW = 352
NODES_PAD = NW * NPW
NB = 16
NCH = NPW // NB
ROWS = NB * 7
IDXW = NPW * 7
IDX_PAD = NW * IDXW
UN = 4
INV7 = float(1.0 / 7.0)


def _pool_body(x_hbm, no_hbm, out_hbm, idx_raw, gbuf, obuf,
               sem_g0, sem_g1, sem_w0, sem_w1):
    wid = lax.axis_index("s") * 2 + lax.axis_index("c")
    node_base = wid * NPW

    pltpu.sync_copy(no_hbm.at[pl.ds(wid * IDXW, IDXW)], idx_raw)

    lane7 = 7 * lax.iota(jnp.int32, 16)
    sem_g = (sem_g0, sem_g1)
    sem_w = (sem_w0, sem_w1)

    def g_desc(b, c, par):
        idxsl = idx_raw.at[pl.ds(c * ROWS, ROWS)]
        return pltpu.make_async_copy(x_hbm.at[b].at[idxsl], gbuf.at[par],
                                     sem_g[par])

    def w_desc(b, c, par):
        rowbase = b * NODES_PAD + node_base + c * NB
        return pltpu.make_async_copy(obuf.at[par],
                                     out_hbm.at[pl.ds(rowbase, NB)],
                                     sem_w[par])

    def compute_chunk(par):
        for oc in range(16):
            t = [lane7 + (112 * oc + k) for k in range(7)]
            rows = [lax.shift_right_logical(tk, 8) for tk in t]
            cols = [lax.bitwise_and(tk, 255) for tk in t]

            def nbody(ni, _):
                for u in range(UN):
                    n = ni * UN + u
                    roff = 7 * n
                    acc = plsc.load_gather(gbuf.at[par], [rows[0] + roff, cols[0]])
                    for k in range(1, 7):
                        acc = acc + plsc.load_gather(
                            gbuf.at[par], [rows[k] + roff, cols[k]])
                    obuf[par, n, pl.ds(oc * 16, 16)] = acc * INV7
                return 0
            lax.fori_loop(0, NB // UN, nbody, 0)

    def batch_body(b, _):
        g_desc(b, 0, 0).start()
        g_desc(b, 1, 1).start()

        def pair_body(i, _):
            for par in (0, 1):
                c = 2 * i + par
                g_desc(b, c, par).wait()

                @pl.when(i >= 1)
                def _():
                    w_desc(b, c - 2, par).wait()

                compute_chunk(par)
                w_desc(b, c, par).start()

                @pl.when(i < NCH // 2 - 1)
                def _():
                    g_desc(b, c + 2, par).start()
            return 0
        lax.fori_loop(0, NCH // 2, pair_body, 0)

        w_desc(b, NCH - 2, 0).wait()
        w_desc(b, NCH - 1, 1).wait()
        return 0
    lax.fori_loop(0, B, batch_body, 0)


@jax.jit
def _pool(x, no_pad):
    mesh = plsc.VectorSubcoreMesh(core_axis_name="c", subcore_axis_name="s")
    f = pl.kernel(
        _pool_body,
        out_type=jax.ShapeDtypeStruct((B * NODES_PAD, C), jnp.float32),
        mesh=mesh,
        compiler_params=pltpu.CompilerParams(
            use_tc_tiling_on_sc=False, needs_layout_passes=False),
        scratch_types=[
            pltpu.VMEM((IDXW,), jnp.int32),
            pltpu.VMEM((2, ROWS, C), jnp.float32),
            pltpu.VMEM((2, NB, C), jnp.float32),
            pltpu.SemaphoreType.DMA,
            pltpu.SemaphoreType.DMA,
            pltpu.SemaphoreType.DMA,
            pltpu.SemaphoreType.DMA,
        ],
    )
    return f(x, no_pad)


def kernel(x, neigh_orders):
    no_pad = jnp.pad(neigh_orders[: M * 7], (0, IDX_PAD - M * 7))
    out = _pool(x, no_pad)
    return out.reshape(B, NODES_PAD, C)[:, :M, :]


# speedup vs baseline: 8.6142x; 2.4826x over previous
import functools

import jax
import jax.numpy as jnp
from jax import lax
from jax.experimental import pallas as pl
from jax.experimental.pallas import tpu as pltpu
from jax.experimental.pallas import tpu_sc as plsc

B, V, C = 4, 40962, 256
M = (V + 6) // 4
NW = 32
NPW = 352
NODES_PAD = NW * NPW
NB = 16
NCH = NPW // NB
ROWS = NB * 7
IDXW = NPW * 7
IDX_PAD = NW * IDXW
UN = 2
INV7 = float(1.0 / 7.0)


def _pool_body(x_hbm, no_hbm, out_hbm, idx_raw, gbuf, obuf,
               sem_g0, sem_g1, sem_w0, sem_w1):
    wid = lax.axis_index("s") * 2 + lax.axis_index("c")
    node_base = wid * NPW

    pltpu.sync_copy(no_hbm.at[pl.ds(wid * IDXW, IDXW)], idx_raw)

    lane7 = 7 * lax.iota(jnp.int32, 16)
    sem_g = (sem_g0, sem_g1)
    sem_w = (sem_w0, sem_w1)

    def g_desc(b, c, par):
        idxsl = idx_raw.at[pl.ds(c * ROWS, ROWS)]
        return pltpu.make_async_copy(x_hbm.at[b].at[idxsl], gbuf.at[par],
                                     sem_g[par])

    def w_desc(b, c, par):
        rowbase = b * NODES_PAD + node_base + c * NB
        return pltpu.make_async_copy(obuf.at[par],
                                     out_hbm.at[pl.ds(rowbase, NB)],
                                     sem_w[par])

    def compute_chunk(par):
        for oc in range(16):
            t = [lane7 + (112 * oc + k) for k in range(7)]
            rows = [lax.shift_right_logical(tk, 8) for tk in t]
            cols = [lax.bitwise_and(tk, 255) for tk in t]

            def nbody(ni, _):
                for u in range(UN):
                    n = ni * UN + u
                    roff = 7 * n
                    acc = plsc.load_gather(gbuf.at[par], [rows[0] + roff, cols[0]])
                    for k in range(1, 7):
                        acc = acc + plsc.load_gather(
                            gbuf.at[par], [rows[k] + roff, cols[k]])
                    obuf[par, n, pl.ds(oc * 16, 16)] = acc * INV7
                return 0
            lax.fori_loop(0, NB // UN, nbody, 0)

    def batch_body(b, _):
        g_desc(b, 0, 0).start()
        g_desc(b, 1, 1).start()

        def pair_body(i, _):
            for par in (0, 1):
                c = 2 * i + par
                g_desc(b, c, par).wait()

                @pl.when(i >= 1)
                def _():
                    w_desc(b, c - 2, par).wait()

                compute_chunk(par)
                w_desc(b, c, par).start()

                @pl.when(i < NCH // 2 - 1)
                def _():
                    g_desc(b, c + 2, par).start()
            return 0
        lax.fori_loop(0, NCH // 2, pair_body, 0)

        w_desc(b, NCH - 2, 0).wait()
        w_desc(b, NCH - 1, 1).wait()
        return 0
    lax.fori_loop(0, B, batch_body, 0)


@jax.jit
def _pool(x, no_pad):
    mesh = plsc.VectorSubcoreMesh(core_axis_name="c", subcore_axis_name="s")
    f = pl.kernel(
        _pool_body,
        out_type=jax.ShapeDtypeStruct((B * NODES_PAD, C), jnp.float32),
        mesh=mesh,
        compiler_params=pltpu.CompilerParams(
            use_tc_tiling_on_sc=True, needs_layout_passes=False),
        scratch_types=[
            pltpu.VMEM((IDXW,), jnp.int32),
            pltpu.VMEM((2, ROWS, C), jnp.float32),
            pltpu.VMEM((2, NB, C), jnp.float32),
            pltpu.SemaphoreType.DMA,
            pltpu.SemaphoreType.DMA,
            pltpu.SemaphoreType.DMA,
            pltpu.SemaphoreType.DMA,
        ],
    )
    return f(x, no_pad)


def kernel(x, neigh_orders):
    no_pad = jnp.pad(neigh_orders[: M * 7], (0, IDX_PAD - M * 7))
    out = _pool(x, no_pad)
    return out.reshape(B, NODES_PAD, C)[:, :M, :]


# speedup vs baseline: 19.6166x; 2.2772x over previous
import functools

import jax
import jax.numpy as jnp
from jax import lax
from jax.experimental import pallas as pl
from jax.experimental.pallas import tpu as pltpu
from jax.experimental.pallas import tpu_sc as plsc

B, V, C = 4, 40962, 256
M = (V + 6) // 4
NW = 32
NPW = 352
NB = 16
NCH = NPW // NB
ROWS = NB * 7
IDXW = NPW * 7
IDX_PAD = NW * IDXW
UN = 2
INV7 = float(1.0 / 7.0)
PART_NS = (M // NB) * NB
PART_ROWS = M - PART_NS


def _pool_body(x_hbm, no_hbm, out_hbm, idx_raw, gbuf, obuf,
               sem_g0, sem_g1, sem_w0, sem_w1):
    wid = lax.axis_index("s") * 2 + lax.axis_index("c")
    node_base = wid * NPW

    pltpu.sync_copy(no_hbm.at[pl.ds(wid * IDXW, IDXW)], idx_raw)

    lane7 = 7 * lax.iota(jnp.int32, 16)
    sem_g = (sem_g0, sem_g1)
    sem_w = (sem_w0, sem_w1)

    def ns_of(c):
        return node_base + c * NB

    def g_desc(b, c, par):
        idxsl = idx_raw.at[pl.ds(c * ROWS, ROWS)]
        return pltpu.make_async_copy(x_hbm.at[b].at[idxsl], gbuf.at[par],
                                     sem_g[par])

    def w_full_desc(b, c, par):
        return pltpu.make_async_copy(obuf.at[par],
                                     out_hbm.at[b].at[pl.ds(ns_of(c), NB)],
                                     sem_w[par])

    def w_part_desc(b, par):
        return pltpu.make_async_copy(obuf.at[par].at[pl.ds(0, PART_ROWS)],
                                     out_hbm.at[b].at[pl.ds(PART_NS, PART_ROWS)],
                                     sem_w[par])

    def w_act(b, c, par, act):
        ns = ns_of(c)

        @pl.when(ns <= M - NB)
        def _():
            act(w_full_desc(b, c, par))

        @pl.when(ns == PART_NS)
        def _():
            act(w_part_desc(b, par))

    def g_act(b, c, par, act):
        @pl.when(ns_of(c) < M)
        def _():
            act(g_desc(b, c, par))

    def compute_chunk(par):
        for oc in range(16):
            t = [lane7 + (112 * oc + k) for k in range(7)]
            rows = [lax.shift_right_logical(tk, 8) for tk in t]
            cols = [lax.bitwise_and(tk, 255) for tk in t]

            def nbody(ni, _):
                for u in range(UN):
                    n = ni * UN + u
                    roff = 7 * n
                    acc = plsc.load_gather(gbuf.at[par], [rows[0] + roff, cols[0]])
                    for k in range(1, 7):
                        acc = acc + plsc.load_gather(
                            gbuf.at[par], [rows[k] + roff, cols[k]])
                    obuf[par, n, pl.ds(oc * 16, 16)] = acc * INV7
                return 0
            lax.fori_loop(0, NB // UN, nbody, 0)

    def batch_body(b, _):
        g_act(b, 0, 0, lambda d: d.start())
        g_act(b, 1, 1, lambda d: d.start())

        def pair_body(i, _):
            for par in (0, 1):
                c = 2 * i + par
                g_act(b, c, par, lambda d: d.wait())

                @pl.when(i >= 1)
                def _():
                    w_act(b, c - 2, par, lambda d: d.wait())

                @pl.when(ns_of(c) < M)
                def _():
                    compute_chunk(par)

                w_act(b, c, par, lambda d: d.start())

                @pl.when(i < NCH // 2 - 1)
                def _():
                    g_act(b, c + 2, par, lambda d: d.start())
            return 0
        lax.fori_loop(0, NCH // 2, pair_body, 0)

        w_act(b, NCH - 2, 0, lambda d: d.wait())
        w_act(b, NCH - 1, 1, lambda d: d.wait())
        return 0
    lax.fori_loop(0, B, batch_body, 0)


@jax.jit
def _pool(x, no_pad):
    mesh = plsc.VectorSubcoreMesh(core_axis_name="c", subcore_axis_name="s")
    f = pl.kernel(
        _pool_body,
        out_type=jax.ShapeDtypeStruct((B, M, C), jnp.float32),
        mesh=mesh,
        compiler_params=pltpu.CompilerParams(
            use_tc_tiling_on_sc=True, needs_layout_passes=False),
        scratch_types=[
            pltpu.VMEM((IDXW,), jnp.int32),
            pltpu.VMEM((2, ROWS, C), jnp.float32),
            pltpu.VMEM((2, NB, C), jnp.float32),
            pltpu.SemaphoreType.DMA,
            pltpu.SemaphoreType.DMA,
            pltpu.SemaphoreType.DMA,
            pltpu.SemaphoreType.DMA,
        ],
    )
    return f(x, no_pad)


def kernel(x, neigh_orders):
    no_pad = jnp.pad(neigh_orders[: M * 7], (0, IDX_PAD - M * 7))
    return _pool(x, no_pad)


# speedup vs baseline: 19.8915x; 1.0140x over previous
import functools

import jax
import jax.numpy as jnp
from jax import lax
from jax.experimental import pallas as pl
from jax.experimental.pallas import tpu as pltpu
from jax.experimental.pallas import tpu_sc as plsc

B, V, C = 4, 40962, 256
M = (V + 6) // 4
NW = 32
NPW = 352
NB = 16
NCH = NPW // NB
ROWS = NB * 7
IDXW = NPW * 7
IDX_PAD = NW * IDXW
UN = 2
INV7 = float(1.0 / 7.0)
PART_NS = (M // NB) * NB
PART_ROWS = M - PART_NS


def _pool_body(x_hbm, no_hbm, out_hbm, idx_raw, gbuf, obuf,
               sem_g0, sem_g1, sem_w0, sem_w1):
    wid = lax.axis_index("s") * 2 + lax.axis_index("c")
    node_base = wid * NPW

    pltpu.sync_copy(no_hbm.at[pl.ds(wid * IDXW, IDXW)], idx_raw)

    lane7 = 7 * lax.iota(jnp.int32, 16)
    sem_g = (sem_g0, sem_g1)
    sem_w = (sem_w0, sem_w1)

    def ns_of(c):
        return node_base + c * NB

    def g_desc(b, c, par):
        idxsl = idx_raw.at[pl.ds(c * ROWS, ROWS)]
        return pltpu.make_async_copy(x_hbm.at[b].at[idxsl],
                                     gbuf.at[par].at[pl.ds(0, ROWS)],
                                     sem_g[par])

    def w_full_desc(b, c, par):
        return pltpu.make_async_copy(obuf.at[par],
                                     out_hbm.at[b].at[pl.ds(ns_of(c), NB)],
                                     sem_w[par])

    def w_part_desc(b, par):
        return pltpu.make_async_copy(obuf.at[par].at[pl.ds(0, PART_ROWS)],
                                     out_hbm.at[b].at[pl.ds(PART_NS, PART_ROWS)],
                                     sem_w[par])

    def w_act(b, c, par, act):
        ns = ns_of(c)

        @pl.when(ns <= M - NB)
        def _():
            act(w_full_desc(b, c, par))

        @pl.when(ns == PART_NS)
        def _():
            act(w_part_desc(b, par))

    def g_act(b, c, par, act):
        @pl.when(ns_of(c) < M)
        def _():
            act(g_desc(b, c, par))

    def compute_chunk(par):
        for oc in range(16):
            t = [lane7 + (112 * oc + k) for k in range(7)]
            rows = [lax.shift_right_logical(tk, 8) for tk in t]
            cols = [lax.bitwise_and(tk, 255) for tk in t]

            def nbody(ni, _):
                for u in range(UN):
                    n = ni * UN + u
                    roff = 7 * n
                    g = [plsc.load_gather(gbuf.at[par],
                                          [rows[k] + roff, cols[k]])
                         for k in range(7)]
                    acc = ((g[0] + g[1]) + (g[2] + g[3])) + \
                          ((g[4] + g[5]) + g[6])
                    obuf[par, n, pl.ds(oc * 16, 16)] = acc * INV7
                return 0
            lax.fori_loop(0, NB // UN, nbody, 0)

    def batch_body(b, _):
        g_act(b, 0, 0, lambda d: d.start())
        g_act(b, 1, 1, lambda d: d.start())

        def pair_body(i, _):
            for par in (0, 1):
                c = 2 * i + par
                g_act(b, c, par, lambda d: d.wait())

                @pl.when(i >= 1)
                def _():
                    w_act(b, c - 2, par, lambda d: d.wait())

                @pl.when(ns_of(c) < M)
                def _():
                    compute_chunk(par)

                w_act(b, c, par, lambda d: d.start())

                @pl.when(i < NCH // 2 - 1)
                def _():
                    g_act(b, c + 2, par, lambda d: d.start())
            return 0
        lax.fori_loop(0, NCH // 2, pair_body, 0)

        w_act(b, NCH - 2, 0, lambda d: d.wait())
        w_act(b, NCH - 1, 1, lambda d: d.wait())
        return 0
    lax.fori_loop(0, B, batch_body, 0)


@jax.jit
def _pool(x, no_pad):
    mesh = plsc.VectorSubcoreMesh(core_axis_name="c", subcore_axis_name="s")
    f = pl.kernel(
        _pool_body,
        out_type=jax.ShapeDtypeStruct((B, M, C), jnp.float32),
        mesh=mesh,
        compiler_params=pltpu.CompilerParams(
            use_tc_tiling_on_sc=True, needs_layout_passes=False),
        scratch_types=[
            pltpu.VMEM((IDXW,), jnp.int32),
            pltpu.VMEM((2, ROWS + 1, C), jnp.float32),
            pltpu.VMEM((2, NB, C), jnp.float32),
            pltpu.SemaphoreType.DMA,
            pltpu.SemaphoreType.DMA,
            pltpu.SemaphoreType.DMA,
            pltpu.SemaphoreType.DMA,
        ],
    )
    return f(x, no_pad)


def kernel(x, neigh_orders):
    no_pad = jnp.pad(neigh_orders[: M * 7], (0, IDX_PAD - M * 7))
    return _pool(x, no_pad)
